# baseline (speedup 1.0000x reference)
import functools

import jax
import jax.numpy as jnp
from jax import lax
from jax.experimental import pallas as pl
from jax.experimental.pallas import tpu as pltpu
from jax.experimental.pallas import tpu_sc as plsc

_D = 128
_NT = 32
_CHUNK = 320
_NPAD = _NT * _CHUNK
_TOT = _NPAD + 8
_R = 64
_W = _R + 8


def _tc_a(in_ref, w0_ref, w1_ref, wroot_ref, brg_ref, h0_ref, h1_ref, root_ref):
    f32 = jnp.float32
    xin = in_ref[...]
    h0_ref[...] = jnp.dot(xin, w0_ref[...], preferred_element_type=f32)
    h1_ref[...] = jnp.dot(xin, w1_ref[...], preferred_element_type=f32)
    root_ref[...] = (jnp.dot(xin, wroot_ref[...], preferred_element_type=f32)
                     + brg_ref[...])


def _inv_small(x):
    f = jnp.float32
    return jnp.where(x >= f(4.5), f(0.2),
           jnp.where(x >= f(3.5), f(0.25),
           jnp.where(x >= f(2.5), f(1.0 / 3.0),
           jnp.where(x >= f(1.5), f(0.5), f(1.0)))))


def _sc_body(h0_hbm, h1_hbm, root_hbm, spk_hbm, pos_hbm, x_hbm, agg_hbm,
             h0b, h1b, rootb, spkb, posb, xb, aggb):
    wid = lax.axis_index("s") * 2 + lax.axis_index("c")
    r0 = wid * _CHUNK
    f32 = jnp.float32
    one = jnp.float32(1.0)
    zero = jnp.float32(0.0)

    for c in range(_CHUNK // _R):
        p0 = r0 + c * _R
        pltpu.sync_copy(h0_hbm.at[pl.ds(p0, _W), :], h0b)
        pltpu.sync_copy(h1_hbm.at[pl.ds(p0, _W), :], h1b)
        pltpu.sync_copy(root_hbm.at[pl.ds(p0, _W), :], rootb)
        pltpu.sync_copy(spk_hbm.at[pl.ds(p0, _W)], spkb.at[pl.ds(0, _W)])
        pltpu.sync_copy(pos_hbm.at[pl.ds(p0, _W)], posb.at[pl.ds(0, _W)])

        def x_row(l, _):
            sw = spkb[pl.ds(l - 4, 16)]
            pw = posb[pl.ds(l, 16)]
            S = sw[4]
            p = pw[0]
            s = [sw[4 - k] for k in range(5)]
            v = [jnp.where(p >= f32(k), one, zero) for k in range(5)]
            nv = v[0] + v[1] + v[2] + v[3] + v[4]
            c1 = v[0] * s[0] + v[1] * s[1] + v[2] * s[2] + v[3] * s[3] + v[4] * s[4]
            c0 = nv - c1
            inv_nv = _inv_small(nv)
            inv_c0 = _inv_small(jnp.maximum(c0, one))
            inv_c1 = _inv_small(jnp.maximum(c1, one))
            a = (one - S) * inv_nv
            b = S * inv_c0
            cc = S * inv_c1
            co0 = [v[k] * (a + b * (one - s[k])) for k in range(5)]
            co1 = [v[k] * cc * s[k] for k in range(5)]
            for f in range(_D // 16):
                sl = pl.ds(16 * f, 16)
                acc = rootb[l, sl]
                for k in range(5):
                    acc = (acc + co0[k] * h0b[l - k, sl]
                           + co1[k] * h1b[l - k, sl])
                xb[l, sl] = acc
            return _

        lax.fori_loop(4, _W, x_row, 0, unroll=2)

        def agg_row(l, _):
            p = posb[pl.ds(l, 16)][0]
            v = [jnp.where(p >= f32(k), one, zero) for k in range(1, 5)]
            for f in range(_D // 16):
                sl = pl.ds(16 * f, 16)
                acc = xb[l, sl]
                for k in range(1, 5):
                    acc = acc + v[k - 1] * xb[l - k, sl]
                aggb[l - 8, sl] = acc
            return _

        lax.fori_loop(8, _W, agg_row, 0, unroll=2)

        pltpu.sync_copy(xb.at[pl.ds(8, _R), :], x_hbm.at[pl.ds(p0, _R), :])
        pltpu.sync_copy(aggb, agg_hbm.at[pl.ds(p0, _R), :])


_MC = 1024


def _tc_c(in_ref, x_ref, agg_ref, wgroot_ref, wgrel_ref, bgc_ref,
          wsk_ref, bsk_ref, wc_ref, bc_ref, out_ref):
    f32 = jnp.float32
    b = pl.program_id(0)
    xin = in_ref[pl.ds(b * _MC + 8, _MC), :]
    x2 = (jnp.dot(agg_ref[...], wgrel_ref[...], preferred_element_type=f32)
          + jnp.dot(x_ref[...], wgroot_ref[...], preferred_element_type=f32)
          + bgc_ref[...])
    skip = jnp.dot(xin, wsk_ref[...], preferred_element_type=f32) + bsk_ref[...]
    out_ref[...] = (jnp.dot(x2 + skip, wc_ref[...], preferred_element_type=f32)
                    + bc_ref[...])


def kernel(input, dialog_lengths, speakers, W_rel, W_root, b_rgcn,
           Wgc_root, Wgc_rel, b_gc, W_skip, b_skip, W_cls, b_cls):
    N, D = input.shape
    n_cls = W_cls.shape[1]
    f32 = jnp.float32

    starts = jnp.cumsum(dialog_lengths) - dialog_lengths
    n_ids = jnp.arange(N, dtype=jnp.int32)
    start_n = jnp.max(jnp.where(starts[None, :] <= n_ids[:, None],
                                starts[None, :], 0), axis=1)
    pos = (n_ids - start_n).astype(f32)

    inp_p = jnp.zeros((_TOT, D), input.dtype).at[8:8 + N].set(input)
    spk_p = jnp.zeros((_TOT,), f32).at[8:8 + N].set(speakers.astype(f32))
    pos_p = jnp.zeros((_TOT,), f32).at[8:8 + N].set(pos)

    full = lambda shape: pl.BlockSpec(shape, lambda b: (0,) * len(shape))

    ma = _TOT // 3
    h0, h1, root = pl.pallas_call(
        _tc_a,
        grid=(3,),
        in_specs=[pl.BlockSpec((ma, D), lambda b: (b, 0)),
                  full((D, D)), full((D, D)), full((D, D)), full((1, D))],
        out_specs=[pl.BlockSpec((ma, D), lambda b: (b, 0))] * 3,
        out_shape=[jax.ShapeDtypeStruct((_TOT, D), f32)] * 3,
    )(inp_p, W_rel[0], W_rel[1], W_root, b_rgcn.reshape(1, D))

    mesh = plsc.VectorSubcoreMesh(core_axis_name="c", subcore_axis_name="s")
    sc = pl.kernel(
        _sc_body,
        mesh=mesh,
        out_type=[jax.ShapeDtypeStruct((_NPAD, D), f32),
                  jax.ShapeDtypeStruct((_NPAD, D), f32)],
        scratch_types=[
            pltpu.VMEM((_W, D), f32), pltpu.VMEM((_W, D), f32),
            pltpu.VMEM((_W, D), f32), pltpu.VMEM((_W + 16,), f32),
            pltpu.VMEM((_W + 16,), f32), pltpu.VMEM((_W, D), f32),
            pltpu.VMEM((_R, D), f32),
        ],
    )
    x, agg = sc(h0, h1, root, spk_p, pos_p)

    out = pl.pallas_call(
        _tc_c,
        grid=(_NPAD // _MC,),
        in_specs=[full((_TOT, D)),
                  pl.BlockSpec((_MC, D), lambda b: (b, 0)),
                  pl.BlockSpec((_MC, D), lambda b: (b, 0)),
                  full((D, D)), full((D, D)), full((1, D)),
                  full((D, D)), full((1, D)),
                  full((D, n_cls)), full((1, n_cls))],
        out_specs=pl.BlockSpec((_MC, n_cls), lambda b: (b, 0)),
        out_shape=jax.ShapeDtypeStruct((_NPAD, n_cls), f32),
    )(inp_p, x, agg, Wgc_root, Wgc_rel, b_gc.reshape(1, D),
      W_skip, b_skip.reshape(1, D), W_cls, b_cls.reshape(1, n_cls))
    return out[:N]


# speedup vs baseline: 1.2516x; 1.2516x over previous
import functools

import jax
import jax.numpy as jnp
from jax import lax
from jax.experimental import pallas as pl
from jax.experimental.pallas import tpu as pltpu
from jax.experimental.pallas import tpu_sc as plsc

_D = 128
_NT = 32
_CHUNK = 320
_NPAD = _NT * _CHUNK
_TOT = _NPAD + 8
_R = 64
_W = _R + 8


def _tc_a(in_ref, w0_ref, w1_ref, wroot_ref, brg_ref, h0_ref, h1_ref, root_ref):
    f32 = jnp.float32
    xin = in_ref[...]
    h0_ref[...] = jnp.dot(xin, w0_ref[...], preferred_element_type=f32)
    h1_ref[...] = jnp.dot(xin, w1_ref[...], preferred_element_type=f32)
    root_ref[...] = (jnp.dot(xin, wroot_ref[...], preferred_element_type=f32)
                     + brg_ref[...])


def _inv_small(x):
    f = jnp.float32
    return jnp.where(x >= f(4.5), f(0.2),
           jnp.where(x >= f(3.5), f(0.25),
           jnp.where(x >= f(2.5), f(1.0 / 3.0),
           jnp.where(x >= f(1.5), f(0.5), f(1.0)))))


def _sc_body(h0_hbm, h1_hbm, root_hbm, spk_hbm, pos_hbm, x_hbm, agg_hbm,
             h0b, h1b, rootb, spkb, posb, xb, aggb, coefb):
    wid = lax.axis_index("s") * 2 + lax.axis_index("c")
    r0 = wid * _CHUNK
    one = jnp.float32(1.0)
    zero = jnp.float32(0.0)

    for c in range(_CHUNK // _R):
        p0 = r0 + c * _R
        pltpu.sync_copy(h0_hbm.at[pl.ds(p0, _W), :], h0b)
        pltpu.sync_copy(h1_hbm.at[pl.ds(p0, _W), :], h1b)
        pltpu.sync_copy(root_hbm.at[pl.ds(p0, _W), :], rootb)
        pltpu.sync_copy(spk_hbm.at[pl.ds(p0, _W)], spkb.at[pl.ds(0, _W)])
        pltpu.sync_copy(pos_hbm.at[pl.ds(p0, _W)], posb.at[pl.ds(0, _W)])

        for l0 in (4, 20, 36, 52, 56):
            sv = [spkb[pl.ds(l0 - k, 16)] for k in range(5)]
            pv = posb[pl.ds(l0, 16)]
            v = [jnp.where(pv >= jnp.float32(k), one, zero) for k in range(5)]
            nv = v[0] + v[1] + v[2] + v[3] + v[4]
            c1 = (v[0] * sv[0] + v[1] * sv[1] + v[2] * sv[2]
                  + v[3] * sv[3] + v[4] * sv[4])
            c0 = nv - c1
            inv_nv = _inv_small(nv)
            inv_c0 = _inv_small(jnp.maximum(c0, one))
            inv_c1 = _inv_small(jnp.maximum(c1, one))
            S = sv[0]
            a = (one - S) * inv_nv
            b = S * inv_c0
            cc = S * inv_c1
            dsl = pl.ds(l0, 16)
            for k in range(5):
                coefb[k, dsl] = v[k] * (a + b * (one - sv[k]))
                coefb[5 + k, dsl] = v[k] * cc * sv[k]
            for k in range(1, 5):
                coefb[9 + k, dsl] = v[k]

        for half in range(2):
            sls = [pl.ds(64 * half + 16 * s, 16) for s in range(4)]

            def x_row(l, w):
                w0, w1 = w
                cv = [coefb[j, pl.ds(l, 16)][0] for j in range(10)]
                n0 = tuple(h0b[l, sl] for sl in sls)
                n1 = tuple(h1b[l, sl] for sl in sls)
                for s, sl in enumerate(sls):
                    acc = rootb[l, sl] + cv[0] * n0[s] + cv[5] * n1[s]
                    for k in range(1, 5):
                        acc = (acc + cv[k] * w0[4 - k][s]
                               + cv[5 + k] * w1[4 - k][s])
                    xb[l, sl] = acc
                return ((w0[1], w0[2], w0[3], n0), (w1[1], w1[2], w1[3], n1))

            w0_init = tuple(tuple(h0b[r, sl] for sl in sls) for r in range(4))
            w1_init = tuple(tuple(h1b[r, sl] for sl in sls) for r in range(4))
            lax.fori_loop(4, _W, x_row, (w0_init, w1_init), unroll=2)

        sls8 = [pl.ds(16 * s, 16) for s in range(8)]

        def agg_row(l, wx):
            cv = [coefb[9 + k, pl.ds(l, 16)][0] for k in range(1, 5)]
            nx = tuple(xb[l, sl] for sl in sls8)
            for s in range(8):
                acc = nx[s]
                for k in range(1, 5):
                    acc = acc + cv[k - 1] * wx[4 - k][s]
                aggb[l - 8, sls8[s]] = acc
            return (wx[1], wx[2], wx[3], nx)

        wx_init = tuple(tuple(xb[r, sl] for sl in sls8) for r in range(4, 8))
        lax.fori_loop(8, _W, agg_row, wx_init, unroll=2)

        pltpu.sync_copy(xb.at[pl.ds(8, _R), :], x_hbm.at[pl.ds(p0, _R), :])
        pltpu.sync_copy(aggb, agg_hbm.at[pl.ds(p0, _R), :])


_MC = 1024


def _tc_c(in_ref, x_ref, agg_ref, wgroot_ref, wgrel_ref, bgc_ref,
          wsk_ref, bsk_ref, wc_ref, bc_ref, out_ref):
    f32 = jnp.float32
    b = pl.program_id(0)
    xin = in_ref[pl.ds(b * _MC + 8, _MC), :]
    x2 = (jnp.dot(agg_ref[...], wgrel_ref[...], preferred_element_type=f32)
          + jnp.dot(x_ref[...], wgroot_ref[...], preferred_element_type=f32)
          + bgc_ref[...])
    skip = jnp.dot(xin, wsk_ref[...], preferred_element_type=f32) + bsk_ref[...]
    out_ref[...] = (jnp.dot(x2 + skip, wc_ref[...], preferred_element_type=f32)
                    + bc_ref[...])


def kernel(input, dialog_lengths, speakers, W_rel, W_root, b_rgcn,
           Wgc_root, Wgc_rel, b_gc, W_skip, b_skip, W_cls, b_cls):
    N, D = input.shape
    n_cls = W_cls.shape[1]
    f32 = jnp.float32

    starts = jnp.cumsum(dialog_lengths) - dialog_lengths
    n_ids = jnp.arange(N, dtype=jnp.int32)
    start_n = jnp.max(jnp.where(starts[None, :] <= n_ids[:, None],
                                starts[None, :], 0), axis=1)
    pos = (n_ids - start_n).astype(f32)

    inp_p = jnp.zeros((_TOT, D), input.dtype).at[8:8 + N].set(input)
    spk_p = jnp.zeros((_TOT,), f32).at[8:8 + N].set(speakers.astype(f32))
    pos_p = jnp.zeros((_TOT,), f32).at[8:8 + N].set(pos)

    full = lambda shape: pl.BlockSpec(shape, lambda b: (0,) * len(shape))

    ma = _TOT // 3
    h0, h1, root = pl.pallas_call(
        _tc_a,
        grid=(3,),
        in_specs=[pl.BlockSpec((ma, D), lambda b: (b, 0)),
                  full((D, D)), full((D, D)), full((D, D)), full((1, D))],
        out_specs=[pl.BlockSpec((ma, D), lambda b: (b, 0))] * 3,
        out_shape=[jax.ShapeDtypeStruct((_TOT, D), f32)] * 3,
    )(inp_p, W_rel[0], W_rel[1], W_root, b_rgcn.reshape(1, D))

    mesh = plsc.VectorSubcoreMesh(core_axis_name="c", subcore_axis_name="s")
    sc = pl.kernel(
        _sc_body,
        mesh=mesh,
        out_type=[jax.ShapeDtypeStruct((_NPAD, D), f32),
                  jax.ShapeDtypeStruct((_NPAD, D), f32)],
        scratch_types=[
            pltpu.VMEM((_W, D), f32), pltpu.VMEM((_W, D), f32),
            pltpu.VMEM((_W, D), f32), pltpu.VMEM((_W + 16,), f32),
            pltpu.VMEM((_W + 16,), f32), pltpu.VMEM((_W, D), f32),
            pltpu.VMEM((_R, D), f32), pltpu.VMEM((14, _W + 16), f32),
        ],
    )
    x, agg = sc(h0, h1, root, spk_p, pos_p)

    out = pl.pallas_call(
        _tc_c,
        grid=(_NPAD // _MC,),
        in_specs=[full((_TOT, D)),
                  pl.BlockSpec((_MC, D), lambda b: (b, 0)),
                  pl.BlockSpec((_MC, D), lambda b: (b, 0)),
                  full((D, D)), full((D, D)), full((1, D)),
                  full((D, D)), full((1, D)),
                  full((D, n_cls)), full((1, n_cls))],
        out_specs=pl.BlockSpec((_MC, n_cls), lambda b: (b, 0)),
        out_shape=jax.ShapeDtypeStruct((_NPAD, n_cls), f32),
    )(inp_p, x, agg, Wgc_root, Wgc_rel, b_gc.reshape(1, D),
      W_skip, b_skip.reshape(1, D), W_cls, b_cls.reshape(1, n_cls))
    return out[:N]
